# baseline (speedup 1.0000x reference)
import functools

import jax
import jax.numpy as jnp
import numpy as np
from jax.experimental import pallas as pl
from jax.experimental.pallas import tpu as pltpu
from jax.experimental.pallas import tpu_sc as plsc

BATCH = 1
SEQ = 2048
DIM = 768
HEADS = 12
DIM_HEAD = 64
SLIDING = 64
CBS = 16
SBS = 16
NUM_SEL = 4
NUM_MEM = 4
SCALE = DIM_HEAD ** -0.5
NBLK = SEQ // CBS
CTX = NUM_MEM + NBLK
NEG = -1e30
INNER = HEADS * DIM_HEAD
CDIM = CBS * DIM_HEAD

BQ = 256
BK = 256
NT = SEQ // BK
GQ = SEQ // BQ
NB_T = BK // SBS
SW = BQ + 2 * SLIDING


def _tables():
    inv = 1.0 / (10000.0 ** (np.arange(0, DIM_HEAD, 2, dtype=np.float64) / DIM_HEAD))
    f = np.arange(SEQ, dtype=np.float64)[:, None] * inv[None, :]
    f = np.repeat(f, 2, axis=-1)
    cos = np.cos(f.astype(np.float32)).astype(np.float32)
    sin = np.sin(f.astype(np.float32)).astype(np.float32)
    P = np.zeros((DIM_HEAD, DIM_HEAD), np.float32)
    for k in range(DIM_HEAD // 2):
        P[2 * k + 1, 2 * k] = -1.0
        P[2 * k, 2 * k + 1] = 1.0
    E16 = np.zeros((NB_T, BK), np.float32)
    for b in range(NB_T):
        E16[b, b * SBS:(b + 1) * SBS] = 1.0
    return jnp.asarray(cos), jnp.asarray(sin), jnp.asarray(P), jnp.asarray(E16)


def _gate_selectors():
    sels = []
    for j in range(3):
        G = np.zeros((3 * HEADS, DIM), np.float32)
        for h in range(HEADS):
            G[h * 3 + j, h * DIM_HEAD:(h + 1) * DIM_HEAD] = 1.0
        sels.append(jnp.asarray(G))
    return sels


def _qkv_kernel(x_ref, gn_ref, wq_ref, wk_ref, wv_ref, wcomb_ref,
                q_ref, k_ref, v_ref, gate_ref):
    x = x_ref[...]
    ms = jnp.mean(x * x, axis=-1, keepdims=True)
    xn = x * jax.lax.rsqrt(ms + jnp.finfo(jnp.float32).eps) * gn_ref[...]
    q_ref[...] = jnp.dot(xn, wq_ref[...], preferred_element_type=jnp.float32)
    k_ref[...] = jnp.dot(xn, wk_ref[...], preferred_element_type=jnp.float32)
    v_ref[...] = jnp.dot(xn, wv_ref[...], preferred_element_type=jnp.float32)
    gate_ref[...] = jnp.dot(xn, wcomb_ref[...], preferred_element_type=jnp.float32)


_SC_NC, _SC_NS = 2, 16
_SC_NW = _SC_NC * _SC_NS
_SC_ROWS = HEADS * NBLK
_SC_BPW = _SC_ROWS // _SC_NW


def _sc_bias_idx():
    r = np.arange(_SC_ROWS, dtype=np.int32)
    return jnp.asarray((r // NBLK).reshape(_SC_NW, _SC_BPW))


def _sc_bias_kernel(kt_hbm, vt_hbm, idx_hbm, ko_hbm, vo_hbm, idx_v, rows_v, sem):
    wid = jax.lax.axis_index("s") * _SC_NC + jax.lax.axis_index("c")
    base = wid * _SC_BPW
    pltpu.sync_copy(idx_hbm.at[wid], idx_v)
    pltpu.async_copy(kt_hbm.at[idx_v], rows_v, sem).wait()
    pltpu.sync_copy(rows_v, ko_hbm.at[pl.ds(base, _SC_BPW)])
    pltpu.async_copy(vt_hbm.at[idx_v], rows_v, sem).wait()
    pltpu.sync_copy(rows_v, vo_hbm.at[pl.ds(base, _SC_BPW)])


def _sc_bias_expand(k_intra, v_intra):
    run = functools.partial(
        pl.kernel,
        mesh=plsc.VectorSubcoreMesh(core_axis_name="c", subcore_axis_name="s"),
        out_type=[
            jax.ShapeDtypeStruct((_SC_ROWS, CDIM), jnp.float32),
            jax.ShapeDtypeStruct((_SC_ROWS, CDIM), jnp.float32),
        ],
        scratch_types=[
            pltpu.VMEM((_SC_BPW,), jnp.int32),
            pltpu.VMEM((_SC_BPW, CDIM), jnp.float32),
            pltpu.SemaphoreType.DMA,
        ],
    )(_sc_bias_kernel)
    return run(k_intra.reshape(HEADS, CDIM), v_intra.reshape(HEADS, CDIM),
               _sc_bias_idx())


def _cmlp_kernel(kc_ref, vc_ref, kin_ref, vin_ref, wk1_ref, bk1_ref, wk2_ref,
                 bk2_ref, wv1_ref, bv1_ref, wv2_ref, bv2_ref, ck_ref, cv_ref):
    kc = kc_ref[...] + kin_ref[...]
    vc = vc_ref[...] + vin_ref[...]
    h1 = jnp.maximum(jnp.dot(kc, wk1_ref[...], preferred_element_type=jnp.float32) + bk1_ref[...], 0.0)
    ck_ref[...] = jnp.dot(h1, wk2_ref[...], preferred_element_type=jnp.float32) + bk2_ref[...]
    h2 = jnp.maximum(jnp.dot(vc, wv1_ref[...], preferred_element_type=jnp.float32) + bv1_ref[...], 0.0)
    cv_ref[...] = jnp.dot(h2, wv2_ref[...], preferred_element_type=jnp.float32) + bv2_ref[...]


def _ce_kernel(q_ref, k_ref, v_ref, ck_ref, cv_ref, cos_ref, sin_ref,
               p64_ref, e16_ref, co_ref, fo_ref, lo_ref,
               rk_ref, vext_ref, mk_ref):
    g = pl.program_id(1)
    p64 = p64_ref[...]

    @pl.when(g == 0)
    def _():
        lane = jax.lax.broadcasted_iota(jnp.int32, (SEQ, DIM_HEAD), 1)
        ones_col = jnp.where(lane == 0, 1.0, 0.0)
        for hh in range(2):
            sl = slice(hh * DIM_HEAD, (hh + 1) * DIM_HEAD)
            kk = k_ref[:, sl]
            rk = kk * cos_ref[...] + jnp.dot(
                kk, p64, preferred_element_type=jnp.float32) * sin_ref[...]
            rk_ref[hh] = rk
            vext_ref[hh, :, :DIM_HEAD] = v_ref[:, sl]
            vext_ref[hh, :, DIM_HEAD:] = ones_col
            mk_ref[0:1, hh:hh + 1] = jnp.max(
                jnp.sum(rk * rk, axis=-1, keepdims=True), axis=0, keepdims=True)

    rowc = g * BQ + jax.lax.broadcasted_iota(jnp.int32, (BQ, CTX), 0)
    colc = jax.lax.broadcasted_iota(jnp.int32, (BQ, CTX), 1)
    ckseq = jnp.where(colc < NUM_MEM, -1, (colc - NUM_MEM + 1) * CBS - 1)
    cmask = ckseq < rowc
    cosq = cos_ref[pl.ds(g * BQ, BQ), :]
    sinq = sin_ref[pl.ds(g * BQ, BQ), :]
    qpos_r = g * BQ + jax.lax.broadcasted_iota(jnp.int32, (BQ, 1), 0)
    own_w = qpos_r // SBS
    e16 = e16_ref[...]
    colb = jax.lax.broadcasted_iota(jnp.int32, (BQ, NB_T), 1)
    start = pl.multiple_of(jnp.maximum(g * BQ - 2 * SLIDING, 0), 2 * SLIDING)
    kpos2 = start + jax.lax.broadcasted_iota(jnp.int32, (BQ, SW), 1)
    qpos2 = g * BQ + jax.lax.broadcasted_iota(jnp.int32, (BQ, SW), 0)
    causal = kpos2 <= qpos2
    band = causal & (qpos2 - kpos2 <= SLIDING)

    for hh in range(2):
        sl = slice(hh * DIM_HEAD, (hh + 1) * DIM_HEAD)
        qb = q_ref[:, sl]

        ck = ck_ref[hh]
        cv = cv_ref[hh]
        csim = jax.lax.dot_general(qb, ck, (((1,), (1,)), ((), ())),
                                   preferred_element_type=jnp.float32) * SCALE
        csim = jnp.where(cmask, csim, NEG)
        mC = jnp.max(csim, axis=-1, keepdims=True)
        eC = jnp.exp(csim - mC)
        p = eC / jnp.sum(eC, axis=-1, keepdims=True)
        co_ref[:, sl] = jnp.dot(p, cv, preferred_element_type=jnp.float32)

        penc = (jax.lax.bitcast_convert_type(p, jnp.int32) & jnp.int32(~0xFF)) | (255 - colc)
        work = jnp.where(colc >= NUM_MEM, penc, -1)
        sel_i = []
        sel_ok = []
        for _ in range(NUM_SEL):
            m = jnp.max(work, axis=-1, keepdims=True)
            work = jnp.where(work == m, -1, work)
            sel_i.append(255 - (m & 0xFF) - NUM_MEM)
            vf = jax.lax.bitcast_convert_type(m & jnp.int32(~0xFF), jnp.float32)
            sel_ok.append(vf > 1e-10)

        rq = (qb * cosq + jnp.dot(qb, p64, preferred_element_type=jnp.float32)
              * sinq) * SCALE
        nq = jnp.sqrt(jnp.sum(rq * rq, axis=-1, keepdims=True))
        m0 = nq * jnp.sqrt(mk_ref[0:1, hh:hh + 1])

        acc = jnp.zeros((BQ, 2 * DIM_HEAD), jnp.float32)
        for t in range(NT):
            kt = rk_ref[hh, t * BK:(t + 1) * BK, :]
            vt = vext_ref[hh, t * BK:(t + 1) * BK, :]
            s = jax.lax.dot_general(rq, kt, (((1,), (1,)), ((), ())),
                                    preferred_element_type=jnp.float32)
            wb = jnp.zeros((BQ, NB_T), jnp.float32)
            jbb = t * NB_T + colb
            for si in range(NUM_SEL):
                wb += ((sel_i[si] == jbb) & sel_ok[si]).astype(jnp.float32)
            w = jnp.dot(wb, e16, preferred_element_type=jnp.float32)
            pt = w * jnp.exp(s - m0)
            acc = acc + jnp.dot(pt, vt, preferred_element_type=jnp.float32)

        ks = rk_ref[hh, pl.ds(start, SW), :]
        vs = vext_ref[hh, pl.ds(start, SW), :]
        bsim = jax.lax.dot_general(rq, ks, (((1,), (1,)), ((), ())),
                                   preferred_element_type=jnp.float32)
        eb = jnp.exp(bsim - m0)
        e_sl = jnp.where(band, eb, 0.0)
        sl_ext = jnp.dot(e_sl, vs, preferred_element_type=jnp.float32)
        lo_ref[:, sl] = sl_ext[:, :DIM_HEAD] / sl_ext[:, DIM_HEAD:DIM_HEAD + 1]
        e_own = jnp.where(causal & ((kpos2 // SBS) == own_w), eb, 0.0)
        acc = acc + jnp.dot(e_own, vs, preferred_element_type=jnp.float32)
        fo_ref[:, sl] = acc[:, :DIM_HEAD] / acc[:, DIM_HEAD:DIM_HEAD + 1]


def _comb_kernel(gate_ref, bcomb_ref, co_ref, fo_ref, lo_ref, g0_ref, g1_ref,
                 g2_ref, wout_ref, out_ref):
    sg = jax.nn.sigmoid(gate_ref[...] + bcomb_ref[...])
    o = (jnp.dot(sg, g0_ref[...], preferred_element_type=jnp.float32) * co_ref[...]
         + jnp.dot(sg, g1_ref[...], preferred_element_type=jnp.float32) * fo_ref[...]
         + jnp.dot(sg, g2_ref[...], preferred_element_type=jnp.float32) * lo_ref[...])
    out_ref[...] = jnp.dot(o, wout_ref[...], preferred_element_type=jnp.float32)


def kernel(inp, g_norm, W_qkv, mem_kv, k_intra, v_intra, Wk1, bk1, Wk2, bk2,
           Wv1, bv1, Wv2, bv2, W_comb, b_comb, W_out):
    n, h, dh = SEQ, HEADS, DIM_HEAD
    x2 = inp.reshape(n, DIM)

    cos, sin, P64, E16 = _tables()
    G0, G1, G2 = _gate_selectors()

    q768, k768, v768, gates = pl.pallas_call(
        _qkv_kernel,
        grid=(GQ,),
        in_specs=[
            pl.BlockSpec((BQ, DIM), lambda i: (i, 0)),
            pl.BlockSpec((1, DIM), lambda i: (0, 0)),
            pl.BlockSpec((DIM, INNER), lambda i: (0, 0)),
            pl.BlockSpec((DIM, INNER), lambda i: (0, 0)),
            pl.BlockSpec((DIM, INNER), lambda i: (0, 0)),
            pl.BlockSpec((DIM, 3 * h), lambda i: (0, 0)),
        ],
        out_specs=[
            pl.BlockSpec((BQ, INNER), lambda i: (i, 0)),
            pl.BlockSpec((BQ, INNER), lambda i: (i, 0)),
            pl.BlockSpec((BQ, INNER), lambda i: (i, 0)),
            pl.BlockSpec((BQ, 3 * h), lambda i: (i, 0)),
        ],
        out_shape=[
            jax.ShapeDtypeStruct((n, INNER), jnp.float32),
            jax.ShapeDtypeStruct((n, INNER), jnp.float32),
            jax.ShapeDtypeStruct((n, INNER), jnp.float32),
            jax.ShapeDtypeStruct((n, 3 * h), jnp.float32),
        ],
    )(x2, g_norm.reshape(1, DIM), W_qkv[:, :INNER], W_qkv[:, INNER:2 * INNER],
      W_qkv[:, 2 * INNER:], W_comb)

    rows = h * NBLK
    brows = rows // 2
    kc_in = k768.reshape(NBLK, CBS, h, dh).transpose(2, 0, 1, 3).reshape(rows, CDIM)
    vc_in = v768.reshape(NBLK, CBS, h, dh).transpose(2, 0, 1, 3).reshape(rows, CDIM)
    kin_full, vin_full = _sc_bias_expand(k_intra, v_intra)
    ck2, cv2 = pl.pallas_call(
        _cmlp_kernel,
        grid=(2,),
        in_specs=[
            pl.BlockSpec((brows, CDIM), lambda i: (i, 0)),
            pl.BlockSpec((brows, CDIM), lambda i: (i, 0)),
            pl.BlockSpec((brows, CDIM), lambda i: (i, 0)),
            pl.BlockSpec((brows, CDIM), lambda i: (i, 0)),
            pl.BlockSpec((CDIM, CDIM), lambda i: (0, 0)),
            pl.BlockSpec((1, CDIM), lambda i: (0, 0)),
            pl.BlockSpec((CDIM, dh), lambda i: (0, 0)),
            pl.BlockSpec((1, dh), lambda i: (0, 0)),
            pl.BlockSpec((CDIM, CDIM), lambda i: (0, 0)),
            pl.BlockSpec((1, CDIM), lambda i: (0, 0)),
            pl.BlockSpec((CDIM, dh), lambda i: (0, 0)),
            pl.BlockSpec((1, dh), lambda i: (0, 0)),
        ],
        out_specs=[
            pl.BlockSpec((brows, dh), lambda i: (i, 0)),
            pl.BlockSpec((brows, dh), lambda i: (i, 0)),
        ],
        out_shape=[
            jax.ShapeDtypeStruct((rows, dh), jnp.float32),
            jax.ShapeDtypeStruct((rows, dh), jnp.float32),
        ],
    )(kc_in, vc_in, kin_full, vin_full,
      Wk1, bk1.reshape(1, CDIM), Wk2, bk2.reshape(1, dh),
      Wv1, bv1.reshape(1, CDIM), Wv2, bv2.reshape(1, dh))

    ck_full = jnp.concatenate(
        (jnp.broadcast_to(mem_kv[0], (h, NUM_MEM, dh)), ck2.reshape(h, NBLK, dh)),
        axis=1)
    cv_full = jnp.concatenate(
        (jnp.broadcast_to(mem_kv[1], (h, NUM_MEM, dh)), cv2.reshape(h, NBLK, dh)),
        axis=1)

    co, fo, lo = pl.pallas_call(
        _ce_kernel,
        grid=(h // 2, GQ),
        in_specs=[
            pl.BlockSpec((BQ, 2 * dh), lambda i, j: (j, i)),
            pl.BlockSpec((n, 2 * dh), lambda i, j: (0, i)),
            pl.BlockSpec((n, 2 * dh), lambda i, j: (0, i)),
            pl.BlockSpec((2, CTX, dh), lambda i, j: (i, 0, 0)),
            pl.BlockSpec((2, CTX, dh), lambda i, j: (i, 0, 0)),
            pl.BlockSpec((n, dh), lambda i, j: (0, 0)),
            pl.BlockSpec((n, dh), lambda i, j: (0, 0)),
            pl.BlockSpec((dh, dh), lambda i, j: (0, 0)),
            pl.BlockSpec((NB_T, BK), lambda i, j: (0, 0)),
        ],
        out_specs=[
            pl.BlockSpec((BQ, 2 * dh), lambda i, j: (j, i)),
            pl.BlockSpec((BQ, 2 * dh), lambda i, j: (j, i)),
            pl.BlockSpec((BQ, 2 * dh), lambda i, j: (j, i)),
        ],
        out_shape=[
            jax.ShapeDtypeStruct((n, INNER), jnp.float32),
            jax.ShapeDtypeStruct((n, INNER), jnp.float32),
            jax.ShapeDtypeStruct((n, INNER), jnp.float32),
        ],
        scratch_shapes=[
            pltpu.VMEM((2, n, dh), jnp.float32),
            pltpu.VMEM((2, n, 2 * dh), jnp.float32),
            pltpu.VMEM((1, 2), jnp.float32),
        ],
    )(q768, k768, v768, ck_full, cv_full, cos, sin, P64, E16)

    out = pl.pallas_call(
        _comb_kernel,
        grid=(GQ,),
        in_specs=[
            pl.BlockSpec((BQ, 3 * h), lambda i: (i, 0)),
            pl.BlockSpec((1, 3 * h), lambda i: (0, 0)),
            pl.BlockSpec((BQ, INNER), lambda i: (i, 0)),
            pl.BlockSpec((BQ, INNER), lambda i: (i, 0)),
            pl.BlockSpec((BQ, INNER), lambda i: (i, 0)),
            pl.BlockSpec((3 * h, DIM), lambda i: (0, 0)),
            pl.BlockSpec((3 * h, DIM), lambda i: (0, 0)),
            pl.BlockSpec((3 * h, DIM), lambda i: (0, 0)),
            pl.BlockSpec((INNER, DIM), lambda i: (0, 0)),
        ],
        out_specs=pl.BlockSpec((BQ, DIM), lambda i: (i, 0)),
        out_shape=jax.ShapeDtypeStruct((n, DIM), jnp.float32),
    )(gates, b_comb.reshape(1, 3 * h), co, fo, lo, G0, G1, G2, W_out)

    return out.reshape(BATCH, n, DIM)


# speedup vs baseline: 1.0038x; 1.0038x over previous
import functools

import jax
import jax.numpy as jnp
import numpy as np
from jax.experimental import pallas as pl
from jax.experimental.pallas import tpu as pltpu
from jax.experimental.pallas import tpu_sc as plsc

BATCH = 1
SEQ = 2048
DIM = 768
HEADS = 12
DIM_HEAD = 64
SLIDING = 64
CBS = 16
SBS = 16
NUM_SEL = 4
NUM_MEM = 4
SCALE = DIM_HEAD ** -0.5
NBLK = SEQ // CBS
CTX = NUM_MEM + NBLK
NEG = -1e30
INNER = HEADS * DIM_HEAD
CDIM = CBS * DIM_HEAD

BQ = 256
BK = 256
NT = SEQ // BK
GQ = SEQ // BQ
NB_T = BK // SBS
SW = BQ + 2 * SLIDING


def _tables():
    inv = 1.0 / (10000.0 ** (np.arange(0, DIM_HEAD, 2, dtype=np.float64) / DIM_HEAD))
    f = np.arange(SEQ, dtype=np.float64)[:, None] * inv[None, :]
    f = np.repeat(f, 2, axis=-1)
    cos = np.cos(f.astype(np.float32)).astype(np.float32)
    sin = np.sin(f.astype(np.float32)).astype(np.float32)
    P = np.zeros((DIM_HEAD, DIM_HEAD), np.float32)
    for k in range(DIM_HEAD // 2):
        P[2 * k + 1, 2 * k] = -1.0
        P[2 * k, 2 * k + 1] = 1.0
    E16 = np.zeros((NB_T, BK), np.float32)
    for b in range(NB_T):
        E16[b, b * SBS:(b + 1) * SBS] = 1.0
    return jnp.asarray(cos), jnp.asarray(sin), jnp.asarray(P), jnp.asarray(E16)


def _gate_selectors():
    sels = []
    for j in range(3):
        G = np.zeros((3 * HEADS, DIM), np.float32)
        for h in range(HEADS):
            G[h * 3 + j, h * DIM_HEAD:(h + 1) * DIM_HEAD] = 1.0
        sels.append(jnp.asarray(G))
    return sels


def _qkv_kernel(x_ref, gn_ref, wq_ref, wk_ref, wv_ref, wcomb_ref,
                q_ref, k_ref, v_ref, gate_ref):
    x = x_ref[...]
    ms = jnp.mean(x * x, axis=-1, keepdims=True)
    xn = x * jax.lax.rsqrt(ms + jnp.finfo(jnp.float32).eps) * gn_ref[...]
    q_ref[...] = jnp.dot(xn, wq_ref[...], preferred_element_type=jnp.float32)
    k_ref[...] = jnp.dot(xn, wk_ref[...], preferred_element_type=jnp.float32)
    v_ref[...] = jnp.dot(xn, wv_ref[...], preferred_element_type=jnp.float32)
    gate_ref[...] = jnp.dot(xn, wcomb_ref[...], preferred_element_type=jnp.float32)


_SC_NC, _SC_NS = 2, 16
_SC_NW = _SC_NC * _SC_NS
_SC_ROWS = HEADS * NBLK
_SC_BPW = _SC_ROWS // _SC_NW


def _sc_bias_idx():
    r = np.arange(_SC_ROWS, dtype=np.int32)
    return jnp.asarray((r // NBLK).reshape(_SC_NW, _SC_BPW))


def _sc_bias_kernel(kt_hbm, vt_hbm, idx_hbm, ko_hbm, vo_hbm, idx_v, krows_v,
                    vrows_v, sem):
    wid = jax.lax.axis_index("s") * _SC_NC + jax.lax.axis_index("c")
    base = wid * _SC_BPW
    pltpu.sync_copy(idx_hbm.at[wid], idx_v)
    ck = pltpu.async_copy(kt_hbm.at[idx_v], krows_v, sem)
    cv = pltpu.async_copy(vt_hbm.at[idx_v], vrows_v, sem)
    ck.wait()
    cv.wait()
    pltpu.sync_copy(krows_v, ko_hbm.at[pl.ds(base, _SC_BPW)])
    pltpu.sync_copy(vrows_v, vo_hbm.at[pl.ds(base, _SC_BPW)])


def _sc_bias_expand(k_intra, v_intra):
    run = functools.partial(
        pl.kernel,
        mesh=plsc.VectorSubcoreMesh(core_axis_name="c", subcore_axis_name="s"),
        out_type=[
            jax.ShapeDtypeStruct((_SC_ROWS, CDIM), jnp.float32),
            jax.ShapeDtypeStruct((_SC_ROWS, CDIM), jnp.float32),
        ],
        scratch_types=[
            pltpu.VMEM((_SC_BPW,), jnp.int32),
            pltpu.VMEM((_SC_BPW, CDIM), jnp.float32),
            pltpu.VMEM((_SC_BPW, CDIM), jnp.float32),
            pltpu.SemaphoreType.DMA,
        ],
    )(_sc_bias_kernel)
    return run(k_intra.reshape(HEADS, CDIM), v_intra.reshape(HEADS, CDIM),
               _sc_bias_idx())


def _cmlp_kernel(kc_ref, vc_ref, kin_ref, vin_ref, wk1_ref, bk1_ref, wk2_ref,
                 bk2_ref, wv1_ref, bv1_ref, wv2_ref, bv2_ref, ck_ref, cv_ref):
    kc = kc_ref[...] + kin_ref[...]
    vc = vc_ref[...] + vin_ref[...]
    h1 = jnp.maximum(jnp.dot(kc, wk1_ref[...], preferred_element_type=jnp.float32) + bk1_ref[...], 0.0)
    ck_ref[...] = jnp.dot(h1, wk2_ref[...], preferred_element_type=jnp.float32) + bk2_ref[...]
    h2 = jnp.maximum(jnp.dot(vc, wv1_ref[...], preferred_element_type=jnp.float32) + bv1_ref[...], 0.0)
    cv_ref[...] = jnp.dot(h2, wv2_ref[...], preferred_element_type=jnp.float32) + bv2_ref[...]


def _ce_kernel(q_ref, k_ref, v_ref, ck_ref, cv_ref, cos_ref, sin_ref,
               p64_ref, e16_ref, co_ref, fo_ref, lo_ref,
               rk_ref, vext_ref, mk_ref):
    g = pl.program_id(1)
    p64 = p64_ref[...]

    @pl.when(g == 0)
    def _():
        lane = jax.lax.broadcasted_iota(jnp.int32, (SEQ, DIM_HEAD), 1)
        ones_col = jnp.where(lane == 0, 1.0, 0.0)
        for hh in range(2):
            sl = slice(hh * DIM_HEAD, (hh + 1) * DIM_HEAD)
            kk = k_ref[:, sl]
            rk = kk * cos_ref[...] + jnp.dot(
                kk, p64, preferred_element_type=jnp.float32) * sin_ref[...]
            rk_ref[hh] = rk
            vext_ref[hh, :, :DIM_HEAD] = v_ref[:, sl]
            vext_ref[hh, :, DIM_HEAD:] = ones_col
            mk_ref[0:1, hh:hh + 1] = jnp.max(
                jnp.sum(rk * rk, axis=-1, keepdims=True), axis=0, keepdims=True)

    rowc = g * BQ + jax.lax.broadcasted_iota(jnp.int32, (BQ, CTX), 0)
    colc = jax.lax.broadcasted_iota(jnp.int32, (BQ, CTX), 1)
    ckseq = jnp.where(colc < NUM_MEM, -1, (colc - NUM_MEM + 1) * CBS - 1)
    cmask = ckseq < rowc
    cosq = cos_ref[pl.ds(g * BQ, BQ), :]
    sinq = sin_ref[pl.ds(g * BQ, BQ), :]
    qpos_r = g * BQ + jax.lax.broadcasted_iota(jnp.int32, (BQ, 1), 0)
    own_w = qpos_r // SBS
    e16 = e16_ref[...]
    colb = jax.lax.broadcasted_iota(jnp.int32, (BQ, NB_T), 1)
    start = pl.multiple_of(jnp.maximum(g * BQ - 2 * SLIDING, 0), 2 * SLIDING)
    kpos2 = start + jax.lax.broadcasted_iota(jnp.int32, (BQ, SW), 1)
    qpos2 = g * BQ + jax.lax.broadcasted_iota(jnp.int32, (BQ, SW), 0)
    causal = kpos2 <= qpos2
    band = causal & (qpos2 - kpos2 <= SLIDING)

    for hh in range(2):
        sl = slice(hh * DIM_HEAD, (hh + 1) * DIM_HEAD)
        qb = q_ref[:, sl]

        ck = ck_ref[hh]
        cv = cv_ref[hh]
        csim = jax.lax.dot_general(qb, ck, (((1,), (1,)), ((), ())),
                                   preferred_element_type=jnp.float32) * SCALE
        csim = jnp.where(cmask, csim, NEG)
        mC = jnp.max(csim, axis=-1, keepdims=True)
        eC = jnp.exp(csim - mC)
        p = eC / jnp.sum(eC, axis=-1, keepdims=True)
        co_ref[:, sl] = jnp.dot(p, cv, preferred_element_type=jnp.float32)

        penc = (jax.lax.bitcast_convert_type(p, jnp.int32) & jnp.int32(~0xFF)) | (255 - colc)
        work = jnp.where(colc >= NUM_MEM, penc, -1)
        sel_i = []
        sel_ok = []
        for _ in range(NUM_SEL):
            m = jnp.max(work, axis=-1, keepdims=True)
            work = jnp.where(work == m, -1, work)
            sel_i.append(255 - (m & 0xFF) - NUM_MEM)
            vf = jax.lax.bitcast_convert_type(m & jnp.int32(~0xFF), jnp.float32)
            sel_ok.append(vf > 1e-10)

        rq = (qb * cosq + jnp.dot(qb, p64, preferred_element_type=jnp.float32)
              * sinq) * SCALE
        nq = jnp.sqrt(jnp.sum(rq * rq, axis=-1, keepdims=True))
        m0 = nq * jnp.sqrt(mk_ref[0:1, hh:hh + 1])

        acc = jnp.zeros((BQ, 2 * DIM_HEAD), jnp.float32)
        for t in range(NT):
            kt = rk_ref[hh, t * BK:(t + 1) * BK, :]
            vt = vext_ref[hh, t * BK:(t + 1) * BK, :]
            s = jax.lax.dot_general(rq, kt, (((1,), (1,)), ((), ())),
                                    preferred_element_type=jnp.float32)
            wb = jnp.zeros((BQ, NB_T), jnp.float32)
            jbb = t * NB_T + colb
            for si in range(NUM_SEL):
                wb += ((sel_i[si] == jbb) & sel_ok[si]).astype(jnp.float32)
            w = jnp.dot(wb, e16, preferred_element_type=jnp.float32)
            pt = w * jnp.exp(s - m0)
            acc = acc + jnp.dot(pt, vt, preferred_element_type=jnp.float32)

        ks = rk_ref[hh, pl.ds(start, SW), :]
        vs = vext_ref[hh, pl.ds(start, SW), :]
        bsim = jax.lax.dot_general(rq, ks, (((1,), (1,)), ((), ())),
                                   preferred_element_type=jnp.float32)
        eb = jnp.exp(bsim - m0)
        e_sl = jnp.where(band, eb, 0.0)
        sl_ext = jnp.dot(e_sl, vs, preferred_element_type=jnp.float32)
        lo_ref[:, sl] = sl_ext[:, :DIM_HEAD] / sl_ext[:, DIM_HEAD:DIM_HEAD + 1]
        e_own = jnp.where(causal & ((kpos2 // SBS) == own_w), eb, 0.0)
        acc = acc + jnp.dot(e_own, vs, preferred_element_type=jnp.float32)
        fo_ref[:, sl] = acc[:, :DIM_HEAD] / acc[:, DIM_HEAD:DIM_HEAD + 1]


def _comb_kernel(gate_ref, bcomb_ref, co_ref, fo_ref, lo_ref, g0_ref, g1_ref,
                 g2_ref, wout_ref, out_ref):
    sg = jax.nn.sigmoid(gate_ref[...] + bcomb_ref[...])
    o = (jnp.dot(sg, g0_ref[...], preferred_element_type=jnp.float32) * co_ref[...]
         + jnp.dot(sg, g1_ref[...], preferred_element_type=jnp.float32) * fo_ref[...]
         + jnp.dot(sg, g2_ref[...], preferred_element_type=jnp.float32) * lo_ref[...])
    out_ref[...] = jnp.dot(o, wout_ref[...], preferred_element_type=jnp.float32)


def kernel(inp, g_norm, W_qkv, mem_kv, k_intra, v_intra, Wk1, bk1, Wk2, bk2,
           Wv1, bv1, Wv2, bv2, W_comb, b_comb, W_out):
    n, h, dh = SEQ, HEADS, DIM_HEAD
    x2 = inp.reshape(n, DIM)

    cos, sin, P64, E16 = _tables()
    G0, G1, G2 = _gate_selectors()

    q768, k768, v768, gates = pl.pallas_call(
        _qkv_kernel,
        grid=(GQ,),
        in_specs=[
            pl.BlockSpec((BQ, DIM), lambda i: (i, 0)),
            pl.BlockSpec((1, DIM), lambda i: (0, 0)),
            pl.BlockSpec((DIM, INNER), lambda i: (0, 0)),
            pl.BlockSpec((DIM, INNER), lambda i: (0, 0)),
            pl.BlockSpec((DIM, INNER), lambda i: (0, 0)),
            pl.BlockSpec((DIM, 3 * h), lambda i: (0, 0)),
        ],
        out_specs=[
            pl.BlockSpec((BQ, INNER), lambda i: (i, 0)),
            pl.BlockSpec((BQ, INNER), lambda i: (i, 0)),
            pl.BlockSpec((BQ, INNER), lambda i: (i, 0)),
            pl.BlockSpec((BQ, 3 * h), lambda i: (i, 0)),
        ],
        out_shape=[
            jax.ShapeDtypeStruct((n, INNER), jnp.float32),
            jax.ShapeDtypeStruct((n, INNER), jnp.float32),
            jax.ShapeDtypeStruct((n, INNER), jnp.float32),
            jax.ShapeDtypeStruct((n, 3 * h), jnp.float32),
        ],
    )(x2, g_norm.reshape(1, DIM), W_qkv[:, :INNER], W_qkv[:, INNER:2 * INNER],
      W_qkv[:, 2 * INNER:], W_comb)

    rows = h * NBLK
    brows = rows // 2
    kc_in = k768.reshape(NBLK, CBS, h, dh).transpose(2, 0, 1, 3).reshape(rows, CDIM)
    vc_in = v768.reshape(NBLK, CBS, h, dh).transpose(2, 0, 1, 3).reshape(rows, CDIM)
    kin_full, vin_full = _sc_bias_expand(k_intra, v_intra)
    ck2, cv2 = pl.pallas_call(
        _cmlp_kernel,
        grid=(2,),
        in_specs=[
            pl.BlockSpec((brows, CDIM), lambda i: (i, 0)),
            pl.BlockSpec((brows, CDIM), lambda i: (i, 0)),
            pl.BlockSpec((brows, CDIM), lambda i: (i, 0)),
            pl.BlockSpec((brows, CDIM), lambda i: (i, 0)),
            pl.BlockSpec((CDIM, CDIM), lambda i: (0, 0)),
            pl.BlockSpec((1, CDIM), lambda i: (0, 0)),
            pl.BlockSpec((CDIM, dh), lambda i: (0, 0)),
            pl.BlockSpec((1, dh), lambda i: (0, 0)),
            pl.BlockSpec((CDIM, CDIM), lambda i: (0, 0)),
            pl.BlockSpec((1, CDIM), lambda i: (0, 0)),
            pl.BlockSpec((CDIM, dh), lambda i: (0, 0)),
            pl.BlockSpec((1, dh), lambda i: (0, 0)),
        ],
        out_specs=[
            pl.BlockSpec((brows, dh), lambda i: (i, 0)),
            pl.BlockSpec((brows, dh), lambda i: (i, 0)),
        ],
        out_shape=[
            jax.ShapeDtypeStruct((rows, dh), jnp.float32),
            jax.ShapeDtypeStruct((rows, dh), jnp.float32),
        ],
    )(kc_in, vc_in, kin_full, vin_full,
      Wk1, bk1.reshape(1, CDIM), Wk2, bk2.reshape(1, dh),
      Wv1, bv1.reshape(1, CDIM), Wv2, bv2.reshape(1, dh))

    ck_full = jnp.concatenate(
        (jnp.broadcast_to(mem_kv[0], (h, NUM_MEM, dh)), ck2.reshape(h, NBLK, dh)),
        axis=1)
    cv_full = jnp.concatenate(
        (jnp.broadcast_to(mem_kv[1], (h, NUM_MEM, dh)), cv2.reshape(h, NBLK, dh)),
        axis=1)

    co, fo, lo = pl.pallas_call(
        _ce_kernel,
        grid=(h // 2, GQ),
        in_specs=[
            pl.BlockSpec((BQ, 2 * dh), lambda i, j: (j, i)),
            pl.BlockSpec((n, 2 * dh), lambda i, j: (0, i)),
            pl.BlockSpec((n, 2 * dh), lambda i, j: (0, i)),
            pl.BlockSpec((2, CTX, dh), lambda i, j: (i, 0, 0)),
            pl.BlockSpec((2, CTX, dh), lambda i, j: (i, 0, 0)),
            pl.BlockSpec((n, dh), lambda i, j: (0, 0)),
            pl.BlockSpec((n, dh), lambda i, j: (0, 0)),
            pl.BlockSpec((dh, dh), lambda i, j: (0, 0)),
            pl.BlockSpec((NB_T, BK), lambda i, j: (0, 0)),
        ],
        out_specs=[
            pl.BlockSpec((BQ, 2 * dh), lambda i, j: (j, i)),
            pl.BlockSpec((BQ, 2 * dh), lambda i, j: (j, i)),
            pl.BlockSpec((BQ, 2 * dh), lambda i, j: (j, i)),
        ],
        out_shape=[
            jax.ShapeDtypeStruct((n, INNER), jnp.float32),
            jax.ShapeDtypeStruct((n, INNER), jnp.float32),
            jax.ShapeDtypeStruct((n, INNER), jnp.float32),
        ],
        scratch_shapes=[
            pltpu.VMEM((2, n, dh), jnp.float32),
            pltpu.VMEM((2, n, 2 * dh), jnp.float32),
            pltpu.VMEM((1, 2), jnp.float32),
        ],
    )(q768, k768, v768, ck_full, cv_full, cos, sin, P64, E16)

    out = pl.pallas_call(
        _comb_kernel,
        grid=(GQ,),
        in_specs=[
            pl.BlockSpec((BQ, 3 * h), lambda i: (i, 0)),
            pl.BlockSpec((1, 3 * h), lambda i: (0, 0)),
            pl.BlockSpec((BQ, INNER), lambda i: (i, 0)),
            pl.BlockSpec((BQ, INNER), lambda i: (i, 0)),
            pl.BlockSpec((BQ, INNER), lambda i: (i, 0)),
            pl.BlockSpec((3 * h, DIM), lambda i: (0, 0)),
            pl.BlockSpec((3 * h, DIM), lambda i: (0, 0)),
            pl.BlockSpec((3 * h, DIM), lambda i: (0, 0)),
            pl.BlockSpec((INNER, DIM), lambda i: (0, 0)),
        ],
        out_specs=pl.BlockSpec((BQ, DIM), lambda i: (i, 0)),
        out_shape=jax.ShapeDtypeStruct((n, DIM), jnp.float32),
    )(gates, b_comb.reshape(1, 3 * h), co, fo, lo, G0, G1, G2, W_out)

    return out.reshape(BATCH, n, DIM)
